# baseline (speedup 1.0000x reference)
import functools

import jax
import jax.numpy as jnp
from jax import lax
from jax.experimental import pallas as pl
from jax.experimental.pallas import tpu as pltpu
from jax.experimental.pallas import tpu_sc as plsc

B = 4096
L = 50
LP = 64
D = 768
LANES = 16
NV = D // LANES
NW = 32
BW = B // NW
GRP = 8
CH = 32
RING = 128
NSLOT = RING // CH
PACK = BW * L + 2 * CH


def _sc_body(ids_hbm, len_hbm, table_hbm, out_hbm,
             ids_v, len_v, packed, ring, oslab, sem):
  cid = lax.axis_index("c")
  sid = lax.axis_index("s")
  wid = sid * 2 + cid
  base = pl.multiple_of(wid * BW, BW)

  pltpu.sync_copy(ids_hbm.at[pl.ds(base, BW)], ids_v)
  pltpu.sync_copy(len_hbm.at[pl.ds(base, BW + LANES)], len_v)

  zero = jnp.zeros((LANES,), jnp.float32)
  izero = jnp.zeros((LANES,), jnp.int32)

  def row_len(b):
    l16 = len_v[pl.ds(b, LANES)]
    return l16[0]

  def compact(b, cum):
    for j in range(LP // LANES):
      v = ids_v[b, pl.ds(LANES * j, LANES)]
      packed[pl.ds(cum + LANES * j, LANES)] = v
    return cum + row_len(b)

  total = lax.fori_loop(0, BW, compact, 0)
  packed[pl.ds(total, LANES)] = izero
  packed[pl.ds(total + LANES, LANES)] = izero

  nch = (total + CH - 1) // CH

  def issue(io, ioff):
    pltpu.async_copy(
        table_hbm.at[packed.at[pl.ds(pl.multiple_of(io * CH, 8), CH)]],
        ring.at[pl.ds(pl.multiple_of(ioff, 8), CH)],
        sem)

  def wait_chunk():
    pltpu.make_async_copy(
        table_hbm.at[pl.ds(0, CH)],
        ring.at[pl.ds(0, CH)],
        sem).wait()

  for i in range(NSLOT):
    @pl.when(i < nch)
    def _():
      issue(i, i * CH)

  io0 = jnp.minimum(nch, NSLOT)

  def process(b, carry):
    cc, dc, io = carry
    lenb = row_len(b)

    tc = jnp.minimum((cc + lenb + CH - 1) // CH, nch)
    for i in range(5):
      @pl.when(i < tc - dc)
      def _():
        wait_chunk()
    dc = jnp.maximum(dc, tc)

    rpos0 = cc - (cc // RING) * RING

    def abody(p, acc):
      rpos = acc[0]
      vals = tuple(acc[1][k] + ring[rpos, pl.ds(LANES * k, LANES)]
                   for k in range(NV))
      rpos = rpos + 1
      rpos = jnp.where(rpos == RING, 0, rpos)
      return (rpos, vals)

    _, acc = lax.fori_loop(0, lenb, abody, (rpos0, (zero,) * NV))
    cc = cc + lenb

    target = jnp.minimum(nch, (cc + RING) // CH)
    for i in range(5):
      @pl.when(io + i < target)
      def _():
        ci = io + i
        slot = ci - (ci // NSLOT) * NSLOT
        issue(ci, slot * CH)
    io = jnp.maximum(io, target)

    i16 = b % GRP
    lsplat = jnp.full((LANES,), jnp.maximum(lenb, 1), jnp.int32)
    invv = jnp.float32(1.0) / lsplat.astype(jnp.float32)
    for k in range(NV):
      oslab[i16, pl.ds(LANES * k, LANES)] = acc[k] * invv

    @pl.when(i16 == GRP - 1)
    def _flush():
      off = pl.multiple_of(base + b - (GRP - 1), GRP)
      pltpu.sync_copy(oslab, out_hbm.at[pl.ds(off, GRP)])

    return (cc, dc, io)

  lax.fori_loop(0, BW, process, (0, 0, io0))


@jax.jit
def _run(ids_pad, seq_len, table):
  mesh = plsc.VectorSubcoreMesh(core_axis_name="c", subcore_axis_name="s")
  f = pl.kernel(
      _sc_body,
      out_type=jax.ShapeDtypeStruct((B, D), jnp.float32),
      mesh=mesh,
      scratch_types=[
          pltpu.VMEM((BW, LP), jnp.int32),
          pltpu.VMEM((BW + LANES,), jnp.int32),
          pltpu.VMEM((PACK,), jnp.int32),
          pltpu.VMEM((RING, D), jnp.float32),
          pltpu.VMEM((GRP, D), jnp.float32),
          pltpu.SemaphoreType.DMA,
      ],
  )
  return f(ids_pad, seq_len, table)


def kernel(seq_ids, seq_len, table):
  ids_pad = jnp.pad(seq_ids.astype(jnp.int32), ((0, 0), (0, LP - L)))
  len_pad = jnp.pad(seq_len.astype(jnp.int32), (0, LANES), constant_values=1)
  return _run(ids_pad, len_pad, table)


# speedup vs baseline: 1.0823x; 1.0823x over previous
import functools

import jax
import jax.numpy as jnp
from jax import lax
from jax.experimental import pallas as pl
from jax.experimental.pallas import tpu as pltpu
from jax.experimental.pallas import tpu_sc as plsc

B = 4096
L = 50
LP = 64
D = 768
LANES = 16
NV = D // LANES
NW = 32
BW = B // NW
GRP = 8
CH = 8
RING = 128
NSLOT = RING // CH
PACK = BW * L + 2 * CH


def _sc_body(ids_hbm, len_hbm, table_hbm, out_hbm,
             ids_v, len_v, packed, ring, oslab, sem):
  cid = lax.axis_index("c")
  sid = lax.axis_index("s")
  wid = sid * 2 + cid
  base = pl.multiple_of(wid * BW, BW)

  pltpu.sync_copy(ids_hbm.at[pl.ds(base, BW)], ids_v)
  pltpu.sync_copy(len_hbm.at[pl.ds(base, BW + LANES)], len_v)

  zero = jnp.zeros((LANES,), jnp.float32)
  izero = jnp.zeros((LANES,), jnp.int32)

  def row_len(b):
    l16 = len_v[pl.ds(b, LANES)]
    return l16[0]

  def compact(b, cum):
    for j in range(LP // LANES):
      v = ids_v[b, pl.ds(LANES * j, LANES)]
      packed[pl.ds(cum + LANES * j, LANES)] = v
    return cum + row_len(b)

  total = lax.fori_loop(0, BW, compact, 0)
  packed[pl.ds(total, LANES)] = izero
  packed[pl.ds(total + LANES, LANES)] = izero

  nch = (total + CH - 1) // CH

  def issue(io, ioff):
    pltpu.async_copy(
        table_hbm.at[packed.at[pl.ds(pl.multiple_of(io * CH, 8), CH)]],
        ring.at[pl.ds(pl.multiple_of(ioff, 8), CH)],
        sem)

  def wait_chunk():
    pltpu.make_async_copy(
        table_hbm.at[pl.ds(0, CH)],
        ring.at[pl.ds(0, CH)],
        sem).wait()

  for i in range(NSLOT):
    @pl.when(i < nch)
    def _():
      issue(i, i * CH)

  io0 = jnp.minimum(nch, NSLOT)

  def process(b, carry):
    cc, dc, io = carry
    lenb = row_len(b)

    tc = jnp.minimum((cc + lenb + CH - 1) // CH, nch)
    for i in range((L - 1) // CH + 2):
      @pl.when(i < tc - dc)
      def _():
        wait_chunk()
    dc = jnp.maximum(dc, tc)

    rpos0 = cc - (cc // RING) * RING

    def abody(p, acc):
      rpos = acc[0]
      vals = tuple(acc[1][k] + ring[rpos, pl.ds(LANES * k, LANES)]
                   for k in range(NV))
      rpos = rpos + 1
      rpos = jnp.where(rpos == RING, 0, rpos)
      return (rpos, vals)

    _, acc = lax.fori_loop(0, lenb, abody, (rpos0, (zero,) * NV))
    cc = cc + lenb

    target = jnp.minimum(nch, (cc + RING) // CH)
    for i in range((L - 1) // CH + 2):
      @pl.when(io + i < target)
      def _():
        ci = io + i
        slot = ci - (ci // NSLOT) * NSLOT
        issue(ci, slot * CH)
    io = jnp.maximum(io, target)

    i16 = b % GRP
    lsplat = jnp.full((LANES,), jnp.maximum(lenb, 1), jnp.int32)
    invv = jnp.float32(1.0) / lsplat.astype(jnp.float32)
    for k in range(NV):
      oslab[i16, pl.ds(LANES * k, LANES)] = acc[k] * invv

    @pl.when(i16 == GRP - 1)
    def _flush():
      off = pl.multiple_of(base + b - (GRP - 1), GRP)
      pltpu.sync_copy(oslab, out_hbm.at[pl.ds(off, GRP)])

    return (cc, dc, io)

  lax.fori_loop(0, BW, process, (0, 0, io0))


@jax.jit
def _run(ids_pad, seq_len, table):
  mesh = plsc.VectorSubcoreMesh(core_axis_name="c", subcore_axis_name="s")
  f = pl.kernel(
      _sc_body,
      out_type=jax.ShapeDtypeStruct((B, D), jnp.float32),
      mesh=mesh,
      scratch_types=[
          pltpu.VMEM((BW, LP), jnp.int32),
          pltpu.VMEM((BW + LANES,), jnp.int32),
          pltpu.VMEM((PACK,), jnp.int32),
          pltpu.VMEM((RING, D), jnp.float32),
          pltpu.VMEM((GRP, D), jnp.float32),
          pltpu.SemaphoreType.DMA,
      ],
  )
  return f(ids_pad, seq_len, table)


def kernel(seq_ids, seq_len, table):
  ids_pad = jnp.pad(seq_ids.astype(jnp.int32), ((0, 0), (0, LP - L)))
  len_pad = jnp.pad(seq_len.astype(jnp.int32), (0, LANES), constant_values=1)
  return _run(ids_pad, len_pad, table)


# speedup vs baseline: 1.0983x; 1.0147x over previous
import functools

import jax
import jax.numpy as jnp
from jax import lax
from jax.experimental import pallas as pl
from jax.experimental.pallas import tpu as pltpu
from jax.experimental.pallas import tpu_sc as plsc

B = 4096
L = 50
LP = 64
D = 768
LANES = 16
NV = D // LANES
NW = 32
BW = B // NW
GRP = 8
CH = 16
RING = 128
NSLOT = RING // CH
PACK = BW * L + 2 * CH


def _sc_body(ids_hbm, len_hbm, table_hbm, out_hbm,
             ids_v, len_v, packed, ring, oslab, sem, fsem):
  cid = lax.axis_index("c")
  sid = lax.axis_index("s")
  wid = sid * 2 + cid
  base = pl.multiple_of(wid * BW, BW)

  pltpu.sync_copy(ids_hbm.at[pl.ds(base, BW)], ids_v)
  pltpu.sync_copy(len_hbm.at[pl.ds(base, BW + LANES)], len_v)

  zero = jnp.zeros((LANES,), jnp.float32)
  izero = jnp.zeros((LANES,), jnp.int32)

  def row_len(b):
    l16 = len_v[pl.ds(b, LANES)]
    return l16[0]

  def compact(b, cum):
    for j in range(LP // LANES):
      v = ids_v[b, pl.ds(LANES * j, LANES)]
      packed[pl.ds(cum + LANES * j, LANES)] = v
    return cum + row_len(b)

  total = lax.fori_loop(0, BW, compact, 0)
  packed[pl.ds(total, LANES)] = izero
  packed[pl.ds(total + LANES, LANES)] = izero

  nch = (total + CH - 1) // CH

  def issue(io, ioff):
    pltpu.async_copy(
        table_hbm.at[packed.at[pl.ds(pl.multiple_of(io * CH, 8), CH)]],
        ring.at[pl.ds(pl.multiple_of(ioff, 8), CH)],
        sem)

  def wait_chunk():
    pltpu.make_async_copy(
        table_hbm.at[pl.ds(0, CH)],
        ring.at[pl.ds(0, CH)],
        sem).wait()

  for i in range(NSLOT):
    @pl.when(i < nch)
    def _():
      issue(i, i * CH)

  io0 = jnp.minimum(nch, NSLOT)

  def process(b, carry):
    cc, dc, io = carry
    lenb = row_len(b)

    @pl.when(jnp.logical_and(b % GRP == 0, b > 0))
    def _wait_flush():
      pltpu.make_async_copy(
          oslab, out_hbm.at[pl.ds(0, GRP)], fsem).wait()

    tc = jnp.minimum((cc + lenb + CH - 1) // CH, nch)
    for i in range(5):
      @pl.when(i < tc - dc)
      def _():
        wait_chunk()
    dc = jnp.maximum(dc, tc)

    rpos0 = cc - (cc // RING) * RING

    def abody(p, acc):
      rpos = acc[0]
      vals = tuple(acc[1][k] + ring[rpos, pl.ds(LANES * k, LANES)]
                   for k in range(NV))
      rpos = rpos + 1
      rpos = jnp.where(rpos == RING, 0, rpos)
      return (rpos, vals)

    _, acc = lax.fori_loop(0, lenb, abody, (rpos0, (zero,) * NV))
    cc = cc + lenb

    target = jnp.minimum(nch, (cc + RING) // CH)
    for i in range(5):
      @pl.when(io + i < target)
      def _():
        ci = io + i
        slot = ci - (ci // NSLOT) * NSLOT
        issue(ci, slot * CH)
    io = jnp.maximum(io, target)

    i16 = b % GRP
    lsplat = jnp.full((LANES,), jnp.maximum(lenb, 1), jnp.int32)
    invv = jnp.float32(1.0) / lsplat.astype(jnp.float32)
    for k in range(NV):
      oslab[i16, pl.ds(LANES * k, LANES)] = acc[k] * invv

    @pl.when(i16 == GRP - 1)
    def _flush():
      off = pl.multiple_of(base + b - (GRP - 1), GRP)
      pltpu.async_copy(oslab, out_hbm.at[pl.ds(off, GRP)], fsem)

    return (cc, dc, io)

  lax.fori_loop(0, BW, process, (0, 0, io0))
  pltpu.make_async_copy(oslab, out_hbm.at[pl.ds(0, GRP)], fsem).wait()


@jax.jit
def _run(ids_pad, seq_len, table):
  mesh = plsc.VectorSubcoreMesh(core_axis_name="c", subcore_axis_name="s")
  f = pl.kernel(
      _sc_body,
      out_type=jax.ShapeDtypeStruct((B, D), jnp.float32),
      mesh=mesh,
      scratch_types=[
          pltpu.VMEM((BW, LP), jnp.int32),
          pltpu.VMEM((BW + LANES,), jnp.int32),
          pltpu.VMEM((PACK,), jnp.int32),
          pltpu.VMEM((RING, D), jnp.float32),
          pltpu.VMEM((GRP, D), jnp.float32),
          pltpu.SemaphoreType.DMA,
          pltpu.SemaphoreType.DMA,
      ],
  )
  return f(ids_pad, seq_len, table)


def kernel(seq_ids, seq_len, table):
  ids_pad = jnp.pad(seq_ids.astype(jnp.int32), ((0, 0), (0, LP - L)))
  len_pad = jnp.pad(seq_len.astype(jnp.int32), (0, LANES), constant_values=1)
  return _run(ids_pad, len_pad, table)
